# baseline (speedup 1.0000x reference)
import functools

import jax
import jax.numpy as jnp
from jax import lax
from jax.experimental import pallas as pl
from jax.experimental.pallas import tpu as pltpu
from jax.experimental.pallas import tpu_sc as plsc

N_USERS = 5000
N_ITEMS = 5000
N = 10000
D = 128
E = 320000
H = 4
S = 50
B = 4096

NC = 2
NS = 16
NW = NC * NS
L = 16

C = 128
K0 = 33
K1 = 124
NCH = 128
NPASS = 2
NCHP = NCH // NPASS
E_PAD = NS * (K0 + K1) * C
NP = 10112
RPT = NP // NS

_mesh = plsc.VectorSubcoreMesh(core_axis_name="c", subcore_axis_name="s",
                               num_cores=NC, num_subcores=NS)


def _spmm_body(src, rows_i, cols_i, vals, out, acc, idx_r, idx_c, vv, rb0,
               sg0):
    cid = lax.axis_index("c")
    sid = lax.axis_index("s")
    wid = sid * NC + cid

    def zrow(i, carry):
        for k in range(D // L):
            rb0[i, pl.ds(k * L, L)] = jnp.zeros((L,), jnp.float32)
        return carry
    lax.fori_loop(0, C, zrow, 0)
    r0 = sid * RPT
    for q in range(RPT // C):
        pltpu.sync_copy(rb0, acc.at[pl.ds(r0 + q * C, C)])
    rem = RPT - (RPT // C) * C
    if rem:
        pltpu.sync_copy(rb0.at[pl.ds(0, rem)],
                        acc.at[pl.ds(r0 + (RPT // C) * C, rem)])
    plsc.subcore_barrier()

    nch_me = jnp.where(cid == 0, K0, K1)
    for p in range(NPASS):
        pltpu.sync_copy(rows_i.at[wid, pl.ds(p * NCHP, NCHP)], idx_r)
        pltpu.sync_copy(cols_i.at[wid, pl.ds(p * NCHP, NCHP)], idx_c)
        pltpu.sync_copy(vals.at[wid, pl.ds(p * NCHP, NCHP)], vv)
        cnt = jnp.minimum(jnp.maximum(nch_me - p * NCHP, 0), NCHP)

        def chunk(j, carry):
            pltpu.async_copy(src.at[idx_c.at[j]], rb0, sg0).wait()

            def scale16(g, c2):
                vchunk = vv[j, pl.ds(g * L, L)]
                for e16 in range(L):
                    e = g * L + e16
                    val = jnp.broadcast_to(vchunk[e16], (L,))
                    for k in range(D // L):
                        sl = (e, pl.ds(k * L, L))
                        rb0[sl] = rb0[sl] * val
                return c2
            lax.fori_loop(0, C // L, scale16, 0)
            pltpu.sync_copy(rb0, acc.at[idx_r.at[j]], add=True)
            return carry
        lax.fori_loop(0, cnt, chunk, 0)

    plsc.subcore_barrier()
    pltpu.sync_copy(acc.at[pl.ds(r0, RPT)], out.at[pl.ds(cid * NP + r0, RPT)])


_sc_spmm = pl.kernel(
    _spmm_body,
    out_type=jax.ShapeDtypeStruct((NC * NP, D), jnp.float32),
    mesh=_mesh,
    scratch_types=[
        pltpu.VMEM_SHARED((NP, D), jnp.float32),
        pltpu.VMEM((NCHP, C), jnp.int32),
        pltpu.VMEM((NCHP, C), jnp.int32),
        pltpu.VMEM((NCHP, C), jnp.float32),
        pltpu.VMEM((C, D), jnp.float32),
        pltpu.SemaphoreType.DMA,
    ],
)


def _make_sc_gather(K, CG):
    def body(src, gidx, out, idxv, buf, sem):
        cid = lax.axis_index("c")
        sid = lax.axis_index("s")
        wid = sid * NC + cid
        for t in range(K):
            pltpu.sync_copy(gidx.at[pl.ds((wid * K + t) * CG, CG)], idxv)
            pltpu.async_copy(src.at[idxv], buf, sem).wait()
            pltpu.sync_copy(buf, out.at[pl.ds((wid * K + t) * CG, CG)])

    return pl.kernel(
        body,
        out_type=jax.ShapeDtypeStruct((NW * K * CG, D), jnp.float32),
        mesh=_mesh,
        scratch_types=[
            pltpu.VMEM((CG,), jnp.int32),
            pltpu.VMEM((CG, D), jnp.float32),
            pltpu.SemaphoreType.DMA,
        ],
    )


_sc_gather_nb = _make_sc_gather(1, 16)
_sc_gather_bpr = _make_sc_gather(3, 128)


def _pre_body(nb, wq, bq, wk, bk, wv, bv, wo, wob, wqk_o, cb_o, nvo_o, bt_o):
    f32 = jnp.float32
    bt = wob[...]
    for h in range(H):
        woh = wo[:, h * D:(h + 1) * D]
        bt = bt + lax.dot_general(bv[h][None, :], woh,
                                  (((1,), (1,)), ((), ())),
                                  preferred_element_type=f32)
        for s2 in range(2):
            nbs = nb[h, s2]
            khat = lax.dot_general(nbs, wk[h], (((1,), (1,)), ((), ())),
                                   preferred_element_type=f32) + bk[h][None, :]
            wqk_o[h, s2] = lax.dot_general(wq[h], khat,
                                           (((0,), (1,)), ((), ())),
                                           preferred_element_type=f32)
            cb_o[h, s2] = lax.dot_general(bq[h][None, :], khat,
                                          (((1,), (1,)), ((), ())),
                                          preferred_element_type=f32)
            nv = lax.dot_general(nbs, wv[h], (((1,), (1,)), ((), ())),
                                 preferred_element_type=f32) + bv[h][None, :]
            nvo_o[h, s2] = lax.dot_general(nv, woh, (((1,), (1,)), ((), ())),
                                           preferred_element_type=f32)
    bt_o[...] = bt


_tc_pre = pl.pallas_call(
    _pre_body,
    out_shape=[
        jax.ShapeDtypeStruct((H, 2, D, S), jnp.float32),
        jax.ShapeDtypeStruct((H, 2, 1, S), jnp.float32),
        jax.ShapeDtypeStruct((H, 2, S, D), jnp.float32),
        jax.ShapeDtypeStruct((1, D), jnp.float32),
    ],
)


BLK = 1000
NBLK = N_USERS // BLK


def _gat_body(x0, x1, wqk, cb, nvo, bt, out):
    f32 = jnp.float32
    x = x0[...] + x1[...]
    acc = jnp.broadcast_to(bt[...], (BLK, D))
    for h in range(H):
        logits = jnp.dot(x, wqk[h, 0], preferred_element_type=f32)
        logits = logits + cb[h, 0, 0][None, :]
        m = jnp.max(logits, axis=1, keepdims=True)
        p = jnp.exp(logits - m)
        attn = p / jnp.sum(p, axis=1, keepdims=True)
        acc = acc + jnp.dot(attn, nvo[h, 0], preferred_element_type=f32)
    out[...] = acc


_tc_gat = pl.pallas_call(
    _gat_body,
    grid=(2, NBLK),
    in_specs=[
        pl.BlockSpec((BLK, D), lambda u, b: (u * NBLK + b, 0)),
        pl.BlockSpec((BLK, D), lambda u, b: (u * NBLK + b, 0)),
        pl.BlockSpec((H, 1, D, S), lambda u, b: (0, u, 0, 0)),
        pl.BlockSpec((H, 1, 1, S), lambda u, b: (0, u, 0, 0)),
        pl.BlockSpec((H, 1, S, D), lambda u, b: (0, u, 0, 0)),
        pl.BlockSpec((1, D), lambda u, b: (0, 0)),
    ],
    out_specs=pl.BlockSpec((BLK, D), lambda u, b: (u * NBLK + b, 0)),
    out_shape=jax.ShapeDtypeStruct((N, D), jnp.float32),
)


ABLK = 2000


def _add_body(a, b, out):
    out[...] = a[...] + b[...]


_tc_add = pl.pallas_call(
    _add_body,
    grid=(N // ABLK,),
    in_specs=[pl.BlockSpec((ABLK, D), lambda i: (i, 0))] * 2,
    out_specs=pl.BlockSpec((ABLK, D), lambda i: (i, 0)),
    out_shape=jax.ShapeDtypeStruct((N, D), jnp.float32),
)


def _mean_body(a, b, c, d, e, out):
    out[...] = (a[...] + b[...] + c[...] + d[...] + e[...]) * 0.25


_tc_mean = pl.pallas_call(
    _mean_body,
    grid=(N // ABLK,),
    in_specs=[pl.BlockSpec((ABLK, D), lambda i: (i, 0))] * 5,
    out_specs=pl.BlockSpec((ABLK, D), lambda i: (i, 0)),
    out_shape=jax.ShapeDtypeStruct((N, D), jnp.float32),
)


def _l2_body(x, out):
    v = x[...]
    out[...] = jnp.sum(jnp.sum(v * v, axis=2), axis=0, keepdims=True)


_tc_l2 = pl.pallas_call(
    _l2_body,
    out_shape=jax.ShapeDtypeStruct((1, B), jnp.float32),
)


def kernel(embedding, wq, bq, wk, bk, wv, bv, w_out_w, w_out_b, feat_index,
           feat_values, adj_index, adj_values, sampled_users, sampled_items,
           users, pos_items, neg_items):
    f32 = jnp.float32
    i32 = jnp.int32
    emb = embedding.astype(f32)

    def _pack(x, dt):
        xp = jnp.concatenate([x.astype(dt), jnp.zeros((E_PAD - E,), dt)])
        a = xp[:NS * K0 * C].reshape(NS, K0, C)
        b = xp[NS * K0 * C:].reshape(NS, K1, C)
        a = jnp.pad(a, ((0, 0), (0, NCH - K0), (0, 0)))
        b = jnp.pad(b, ((0, 0), (0, NCH - K1), (0, 0)))
        return jnp.stack([a, b], axis=1).reshape(NW, NCH, C)

    def _pad_i(x):
        return _pack(x, i32)

    def _pad_f(x):
        return _pack(x, f32)

    fr = _pad_i(feat_index[0])
    fc = _pad_i(feat_index[1])
    fv = _pad_f(feat_values)
    ar = _pad_i(adj_index[0])
    ac = _pad_i(adj_index[1])
    av = _pad_f(adj_values)

    xq_p = _sc_spmm(emb, fr, fc, fv)

    nb_idx = jnp.concatenate([
        sampled_users.reshape(-1).astype(i32),
        sampled_items.reshape(-1).astype(i32) + N_USERS,
        jnp.zeros((NW * 16 - 2 * H * S,), i32),
    ])
    nb_rows = _sc_gather_nb(emb, nb_idx)
    nb = jnp.stack([nb_rows[:H * S].reshape(H, S, D),
                    nb_rows[H * S:2 * H * S].reshape(H, S, D)], axis=1)

    wqk, cb, nvo, bt = _tc_pre(nb, wq.astype(f32), bq.astype(f32),
                               wk.astype(f32), bk.astype(f32), wv.astype(f32),
                               bv.astype(f32), w_out_w.astype(f32),
                               w_out_b.astype(f32).reshape(1, D))

    reps = _tc_gat(xq_p[:N], xq_p[NP:NP + N], wqk, cb, nvo, bt)

    p1 = _sc_spmm(reps, ar, ac, av)
    cur1 = _tc_add(p1[:N], p1[NP:NP + N])
    p2 = _sc_spmm(cur1, ar, ac, av)
    cur2 = _tc_add(p2[:N], p2[NP:NP + N])
    p3 = _sc_spmm(cur2, ar, ac, av)
    finalr = _tc_mean(reps, cur1, cur2, p3[:N], p3[NP:NP + N])

    gidx = jnp.concatenate([
        users.astype(i32),
        pos_items.astype(i32) + N_USERS,
        neg_items.astype(i32) + N_USERS,
    ])
    rep_g = _sc_gather_bpr(finalr, gidx)
    emb_g = _sc_gather_bpr(emb, gidx)
    l2 = _tc_l2(emb_g.reshape(3, B, D))[0]

    return rep_g[:B], rep_g[B:2 * B], rep_g[2 * B:], l2


# speedup vs baseline: 1.1204x; 1.1204x over previous
import functools

import jax
import jax.numpy as jnp
from jax import lax
from jax.experimental import pallas as pl
from jax.experimental.pallas import tpu as pltpu
from jax.experimental.pallas import tpu_sc as plsc

N_USERS = 5000
N_ITEMS = 5000
N = 10000
D = 128
E = 320000
H = 4
S = 50
B = 4096

NC = 2
NS = 16
NW = NC * NS
L = 16

C = 128
K0 = 48
K1 = 109
NCH = 112
NPASS = 2
NCHP = NCH // NPASS
E_PAD = NS * (K0 + K1) * C
NP = 10112
RPT = NP // NS

_mesh = plsc.VectorSubcoreMesh(core_axis_name="c", subcore_axis_name="s",
                               num_cores=NC, num_subcores=NS)


def _spmm_body(src, rows_i, cols_i, vals, out, acc, idx_r, idx_c, vv, rb0,
               sg0):
    cid = lax.axis_index("c")
    sid = lax.axis_index("s")
    wid = sid * NC + cid

    def zrow(i, carry):
        for k in range(D // L):
            rb0[i, pl.ds(k * L, L)] = jnp.zeros((L,), jnp.float32)
        return carry
    lax.fori_loop(0, C, zrow, 0)
    r0 = sid * RPT
    for q in range(RPT // C):
        pltpu.sync_copy(rb0, acc.at[pl.ds(r0 + q * C, C)])
    rem = RPT - (RPT // C) * C
    if rem:
        pltpu.sync_copy(rb0.at[pl.ds(0, rem)],
                        acc.at[pl.ds(r0 + (RPT // C) * C, rem)])
    plsc.subcore_barrier()

    nch_me = jnp.where(cid == 0, K0, K1)
    for p in range(NPASS):
        pltpu.sync_copy(rows_i.at[wid, pl.ds(p * NCHP, NCHP)], idx_r)
        pltpu.sync_copy(cols_i.at[wid, pl.ds(p * NCHP, NCHP)], idx_c)
        pltpu.sync_copy(vals.at[wid, pl.ds(p * NCHP, NCHP)], vv)
        cnt = jnp.minimum(jnp.maximum(nch_me - p * NCHP, 0), NCHP)

        def chunk(j, carry):
            pltpu.async_copy(src.at[idx_c.at[j]], rb0, sg0).wait()

            def scale16(g, c2):
                vchunk = vv[j, pl.ds(g * L, L)]
                for e16 in range(L):
                    e = g * L + e16
                    val = jnp.broadcast_to(vchunk[e16], (L,))
                    for k in range(D // L):
                        sl = (e, pl.ds(k * L, L))
                        rb0[sl] = rb0[sl] * val
                return c2
            lax.fori_loop(0, C // L, scale16, 0)
            pltpu.sync_copy(rb0, acc.at[idx_r.at[j]], add=True)
            return carry
        lax.fori_loop(0, cnt, chunk, 0)

    plsc.subcore_barrier()
    pltpu.sync_copy(acc.at[pl.ds(r0, RPT)], out.at[pl.ds(cid * NP + r0, RPT)])


_sc_spmm = pl.kernel(
    _spmm_body,
    out_type=jax.ShapeDtypeStruct((NC * NP, D), jnp.float32),
    mesh=_mesh,
    scratch_types=[
        pltpu.VMEM_SHARED((NP, D), jnp.float32),
        pltpu.VMEM((NCHP, C), jnp.int32),
        pltpu.VMEM((NCHP, C), jnp.int32),
        pltpu.VMEM((NCHP, C), jnp.float32),
        pltpu.VMEM((C, D), jnp.float32),
        pltpu.SemaphoreType.DMA,
    ],
)


def _make_sc_gather(K, CG):
    def body(src, gidx, out, idxv, buf, sem):
        cid = lax.axis_index("c")
        sid = lax.axis_index("s")
        wid = sid * NC + cid
        for t in range(K):
            pltpu.sync_copy(gidx.at[pl.ds((wid * K + t) * CG, CG)], idxv)
            pltpu.async_copy(src.at[idxv], buf, sem).wait()
            pltpu.sync_copy(buf, out.at[pl.ds((wid * K + t) * CG, CG)])

    return pl.kernel(
        body,
        out_type=jax.ShapeDtypeStruct((NW * K * CG, D), jnp.float32),
        mesh=_mesh,
        scratch_types=[
            pltpu.VMEM((CG,), jnp.int32),
            pltpu.VMEM((CG, D), jnp.float32),
            pltpu.SemaphoreType.DMA,
        ],
    )


_sc_gather_nb = _make_sc_gather(1, 16)
_sc_gather_bpr = _make_sc_gather(3, 128)


def _pre_body(nb, wq, bq, wk, bk, wv, bv, wo, wob, wqk_o, cb_o, nvo_o, bt_o):
    f32 = jnp.float32
    bt = wob[...]
    for h in range(H):
        woh = wo[:, h * D:(h + 1) * D]
        bt = bt + lax.dot_general(bv[h][None, :], woh,
                                  (((1,), (1,)), ((), ())),
                                  preferred_element_type=f32)
        for s2 in range(2):
            nbs = nb[h, s2]
            khat = lax.dot_general(nbs, wk[h], (((1,), (1,)), ((), ())),
                                   preferred_element_type=f32) + bk[h][None, :]
            wqk_o[h, s2] = lax.dot_general(wq[h], khat,
                                           (((0,), (1,)), ((), ())),
                                           preferred_element_type=f32)
            cb_o[h, s2] = lax.dot_general(bq[h][None, :], khat,
                                          (((1,), (1,)), ((), ())),
                                          preferred_element_type=f32)
            nv = lax.dot_general(nbs, wv[h], (((1,), (1,)), ((), ())),
                                 preferred_element_type=f32) + bv[h][None, :]
            nvo_o[h, s2] = lax.dot_general(nv, woh, (((1,), (1,)), ((), ())),
                                           preferred_element_type=f32)
    bt_o[...] = bt


_tc_pre = pl.pallas_call(
    _pre_body,
    out_shape=[
        jax.ShapeDtypeStruct((H, 2, D, S), jnp.float32),
        jax.ShapeDtypeStruct((H, 2, 1, S), jnp.float32),
        jax.ShapeDtypeStruct((H, 2, S, D), jnp.float32),
        jax.ShapeDtypeStruct((1, D), jnp.float32),
    ],
)


BLK = 1000
NBLK = N_USERS // BLK


def _gat_body(x0, x1, wqk, cb, nvo, bt, out):
    f32 = jnp.float32
    x = x0[...] + x1[...]
    acc = jnp.broadcast_to(bt[...], (BLK, D))
    for h in range(H):
        logits = jnp.dot(x, wqk[h, 0], preferred_element_type=f32)
        logits = logits + cb[h, 0, 0][None, :]
        m = jnp.max(logits, axis=1, keepdims=True)
        p = jnp.exp(logits - m)
        attn = p / jnp.sum(p, axis=1, keepdims=True)
        acc = acc + jnp.dot(attn, nvo[h, 0], preferred_element_type=f32)
    out[...] = acc


_tc_gat = pl.pallas_call(
    _gat_body,
    grid=(2, NBLK),
    in_specs=[
        pl.BlockSpec((BLK, D), lambda u, b: (u * NBLK + b, 0)),
        pl.BlockSpec((BLK, D), lambda u, b: (u * NBLK + b, 0)),
        pl.BlockSpec((H, 1, D, S), lambda u, b: (0, u, 0, 0)),
        pl.BlockSpec((H, 1, 1, S), lambda u, b: (0, u, 0, 0)),
        pl.BlockSpec((H, 1, S, D), lambda u, b: (0, u, 0, 0)),
        pl.BlockSpec((1, D), lambda u, b: (0, 0)),
    ],
    out_specs=pl.BlockSpec((BLK, D), lambda u, b: (u * NBLK + b, 0)),
    out_shape=jax.ShapeDtypeStruct((N, D), jnp.float32),
)


ABLK = 2000


def _add_body(a, b, out):
    out[...] = a[...] + b[...]


_tc_add = pl.pallas_call(
    _add_body,
    grid=(N // ABLK,),
    in_specs=[pl.BlockSpec((ABLK, D), lambda i: (i, 0))] * 2,
    out_specs=pl.BlockSpec((ABLK, D), lambda i: (i, 0)),
    out_shape=jax.ShapeDtypeStruct((N, D), jnp.float32),
)


def _mean_body(a, b, c, d, e, out):
    out[...] = (a[...] + b[...] + c[...] + d[...] + e[...]) * 0.25


_tc_mean = pl.pallas_call(
    _mean_body,
    grid=(N // ABLK,),
    in_specs=[pl.BlockSpec((ABLK, D), lambda i: (i, 0))] * 5,
    out_specs=pl.BlockSpec((ABLK, D), lambda i: (i, 0)),
    out_shape=jax.ShapeDtypeStruct((N, D), jnp.float32),
)


def _l2_body(x, out):
    v = x[...]
    out[...] = jnp.sum(jnp.sum(v * v, axis=2), axis=0, keepdims=True)


_tc_l2 = pl.pallas_call(
    _l2_body,
    out_shape=jax.ShapeDtypeStruct((1, B), jnp.float32),
)


def kernel(embedding, wq, bq, wk, bk, wv, bv, w_out_w, w_out_b, feat_index,
           feat_values, adj_index, adj_values, sampled_users, sampled_items,
           users, pos_items, neg_items):
    f32 = jnp.float32
    i32 = jnp.int32
    emb = embedding.astype(f32)

    def _pack(x, dt):
        xp = jnp.concatenate([x.astype(dt), jnp.zeros((E_PAD - E,), dt)])
        a = xp[:NS * K0 * C].reshape(NS, K0, C)
        b = xp[NS * K0 * C:].reshape(NS, K1, C)
        a = jnp.pad(a, ((0, 0), (0, NCH - K0), (0, 0)))
        b = jnp.pad(b, ((0, 0), (0, NCH - K1), (0, 0)))
        return jnp.stack([a, b], axis=1).reshape(NW, NCH, C)

    def _pad_i(x):
        return _pack(x, i32)

    def _pad_f(x):
        return _pack(x, f32)

    fr = _pad_i(feat_index[0])
    fc = _pad_i(feat_index[1])
    fv = _pad_f(feat_values)
    ar = _pad_i(adj_index[0])
    ac = _pad_i(adj_index[1])
    av = _pad_f(adj_values)

    xq_p = _sc_spmm(emb, fr, fc, fv)

    nb_idx = jnp.concatenate([
        sampled_users.reshape(-1).astype(i32),
        sampled_items.reshape(-1).astype(i32) + N_USERS,
        jnp.zeros((NW * 16 - 2 * H * S,), i32),
    ])
    nb_rows = _sc_gather_nb(emb, nb_idx)
    nb = jnp.stack([nb_rows[:H * S].reshape(H, S, D),
                    nb_rows[H * S:2 * H * S].reshape(H, S, D)], axis=1)

    wqk, cb, nvo, bt = _tc_pre(nb, wq.astype(f32), bq.astype(f32),
                               wk.astype(f32), bk.astype(f32), wv.astype(f32),
                               bv.astype(f32), w_out_w.astype(f32),
                               w_out_b.astype(f32).reshape(1, D))

    reps = _tc_gat(xq_p[:N], xq_p[NP:NP + N], wqk, cb, nvo, bt)

    p1 = _sc_spmm(reps, ar, ac, av)
    cur1 = _tc_add(p1[:N], p1[NP:NP + N])
    p2 = _sc_spmm(cur1, ar, ac, av)
    cur2 = _tc_add(p2[:N], p2[NP:NP + N])
    p3 = _sc_spmm(cur2, ar, ac, av)
    finalr = _tc_mean(reps, cur1, cur2, p3[:N], p3[NP:NP + N])

    gidx = jnp.concatenate([
        users.astype(i32),
        pos_items.astype(i32) + N_USERS,
        neg_items.astype(i32) + N_USERS,
    ])
    rep_g = _sc_gather_bpr(finalr, gidx)
    emb_g = _sc_gather_bpr(emb, gidx)
    l2 = _tc_l2(emb_g.reshape(3, B, D))[0]

    return rep_g[:B], rep_g[B:2 * B], rep_g[2 * B:], l2


# speedup vs baseline: 1.2461x; 1.1122x over previous
import functools

import jax
import jax.numpy as jnp
from jax import lax
from jax.experimental import pallas as pl
from jax.experimental.pallas import tpu as pltpu
from jax.experimental.pallas import tpu_sc as plsc

N_USERS = 5000
N_ITEMS = 5000
N = 10000
D = 128
E = 320000
H = 4
S = 50
B = 4096

NC = 2
NS = 16
NW = NC * NS
L = 16

C = 128
K0 = 64
K1 = 93
NCH = 112
NPASS = 2
NCHP = NCH // NPASS
E_PAD = NS * (K0 + K1) * C
NP = 10112
RPT = NP // NS

_mesh = plsc.VectorSubcoreMesh(core_axis_name="c", subcore_axis_name="s",
                               num_cores=NC, num_subcores=NS)


def _spmm_body(src, rows_i, cols_i, vals, out, acc, idx_r, idx_c, vv, rb0,
               sg0):
    cid = lax.axis_index("c")
    sid = lax.axis_index("s")
    wid = sid * NC + cid

    def zrow(i, carry):
        for k in range(D // L):
            rb0[i, pl.ds(k * L, L)] = jnp.zeros((L,), jnp.float32)
        return carry
    lax.fori_loop(0, C, zrow, 0)
    r0 = sid * RPT
    for q in range(RPT // C):
        pltpu.sync_copy(rb0, acc.at[pl.ds(r0 + q * C, C)])
    rem = RPT - (RPT // C) * C
    if rem:
        pltpu.sync_copy(rb0.at[pl.ds(0, rem)],
                        acc.at[pl.ds(r0 + (RPT // C) * C, rem)])
    plsc.subcore_barrier()

    nch_me = jnp.where(cid == 0, K0, K1)
    for p in range(NPASS):
        pltpu.sync_copy(rows_i.at[wid, pl.ds(p * NCHP, NCHP)], idx_r)
        pltpu.sync_copy(cols_i.at[wid, pl.ds(p * NCHP, NCHP)], idx_c)
        pltpu.sync_copy(vals.at[wid, pl.ds(p * NCHP, NCHP)], vv)
        cnt = jnp.minimum(jnp.maximum(nch_me - p * NCHP, 0), NCHP)

        def chunk(j, carry):
            pltpu.async_copy(src.at[idx_c.at[j]], rb0, sg0).wait()

            def scale16(g, c2):
                vchunk = vv[j, pl.ds(g * L, L)]
                for e16 in range(L):
                    e = g * L + e16
                    val = jnp.broadcast_to(vchunk[e16], (L,))
                    for k in range(D // L):
                        sl = (e, pl.ds(k * L, L))
                        rb0[sl] = rb0[sl] * val
                return c2
            lax.fori_loop(0, C // L, scale16, 0)
            pltpu.sync_copy(rb0, acc.at[idx_r.at[j]], add=True)
            return carry
        lax.fori_loop(0, cnt, chunk, 0)

    plsc.subcore_barrier()
    pltpu.sync_copy(acc.at[pl.ds(r0, RPT)], out.at[pl.ds(cid * NP + r0, RPT)])


_sc_spmm = pl.kernel(
    _spmm_body,
    out_type=jax.ShapeDtypeStruct((NC * NP, D), jnp.float32),
    mesh=_mesh,
    scratch_types=[
        pltpu.VMEM_SHARED((NP, D), jnp.float32),
        pltpu.VMEM((NCHP, C), jnp.int32),
        pltpu.VMEM((NCHP, C), jnp.int32),
        pltpu.VMEM((NCHP, C), jnp.float32),
        pltpu.VMEM((C, D), jnp.float32),
        pltpu.SemaphoreType.DMA,
    ],
)


def _make_sc_gather(K, CG):
    def body(src, gidx, out, idxv, buf, sem):
        cid = lax.axis_index("c")
        sid = lax.axis_index("s")
        wid = sid * NC + cid
        for t in range(K):
            pltpu.sync_copy(gidx.at[pl.ds((wid * K + t) * CG, CG)], idxv)
            pltpu.async_copy(src.at[idxv], buf, sem).wait()
            pltpu.sync_copy(buf, out.at[pl.ds((wid * K + t) * CG, CG)])

    return pl.kernel(
        body,
        out_type=jax.ShapeDtypeStruct((NW * K * CG, D), jnp.float32),
        mesh=_mesh,
        scratch_types=[
            pltpu.VMEM((CG,), jnp.int32),
            pltpu.VMEM((CG, D), jnp.float32),
            pltpu.SemaphoreType.DMA,
        ],
    )


_sc_gather_nb = _make_sc_gather(1, 16)
_sc_gather_bpr = _make_sc_gather(3, 128)


def _pre_body(nb, wq, bq, wk, bk, wv, bv, wo, wob, wqk_o, cb_o, nvo_o, bt_o):
    f32 = jnp.float32
    bt = wob[...]
    for h in range(H):
        woh = wo[:, h * D:(h + 1) * D]
        bt = bt + lax.dot_general(bv[h][None, :], woh,
                                  (((1,), (1,)), ((), ())),
                                  preferred_element_type=f32)
        for s2 in range(2):
            nbs = nb[h, s2]
            khat = lax.dot_general(nbs, wk[h], (((1,), (1,)), ((), ())),
                                   preferred_element_type=f32) + bk[h][None, :]
            wqk_o[h, s2] = lax.dot_general(wq[h], khat,
                                           (((0,), (1,)), ((), ())),
                                           preferred_element_type=f32)
            cb_o[h, s2] = lax.dot_general(bq[h][None, :], khat,
                                          (((1,), (1,)), ((), ())),
                                          preferred_element_type=f32)
            nv = lax.dot_general(nbs, wv[h], (((1,), (1,)), ((), ())),
                                 preferred_element_type=f32) + bv[h][None, :]
            nvo_o[h, s2] = lax.dot_general(nv, woh, (((1,), (1,)), ((), ())),
                                           preferred_element_type=f32)
    bt_o[...] = bt


_tc_pre = pl.pallas_call(
    _pre_body,
    out_shape=[
        jax.ShapeDtypeStruct((H, 2, D, S), jnp.float32),
        jax.ShapeDtypeStruct((H, 2, 1, S), jnp.float32),
        jax.ShapeDtypeStruct((H, 2, S, D), jnp.float32),
        jax.ShapeDtypeStruct((1, D), jnp.float32),
    ],
)


BLK = 1000
NBLK = N_USERS // BLK


def _gat_body(x0, x1, wqk, cb, nvo, bt, out):
    f32 = jnp.float32
    x = x0[...] + x1[...]
    acc = jnp.broadcast_to(bt[...], (BLK, D))
    for h in range(H):
        logits = jnp.dot(x, wqk[h, 0], preferred_element_type=f32)
        logits = logits + cb[h, 0, 0][None, :]
        m = jnp.max(logits, axis=1, keepdims=True)
        p = jnp.exp(logits - m)
        attn = p / jnp.sum(p, axis=1, keepdims=True)
        acc = acc + jnp.dot(attn, nvo[h, 0], preferred_element_type=f32)
    out[...] = acc


_tc_gat = pl.pallas_call(
    _gat_body,
    grid=(2, NBLK),
    in_specs=[
        pl.BlockSpec((BLK, D), lambda u, b: (u * NBLK + b, 0)),
        pl.BlockSpec((BLK, D), lambda u, b: (u * NBLK + b, 0)),
        pl.BlockSpec((H, 1, D, S), lambda u, b: (0, u, 0, 0)),
        pl.BlockSpec((H, 1, 1, S), lambda u, b: (0, u, 0, 0)),
        pl.BlockSpec((H, 1, S, D), lambda u, b: (0, u, 0, 0)),
        pl.BlockSpec((1, D), lambda u, b: (0, 0)),
    ],
    out_specs=pl.BlockSpec((BLK, D), lambda u, b: (u * NBLK + b, 0)),
    out_shape=jax.ShapeDtypeStruct((N, D), jnp.float32),
)


ABLK = 2000


def _add_body(a, b, out):
    out[...] = a[...] + b[...]


_tc_add = pl.pallas_call(
    _add_body,
    grid=(N // ABLK,),
    in_specs=[pl.BlockSpec((ABLK, D), lambda i: (i, 0))] * 2,
    out_specs=pl.BlockSpec((ABLK, D), lambda i: (i, 0)),
    out_shape=jax.ShapeDtypeStruct((N, D), jnp.float32),
)


def _mean_body(a, b, c, d, e, out):
    out[...] = (a[...] + b[...] + c[...] + d[...] + e[...]) * 0.25


_tc_mean = pl.pallas_call(
    _mean_body,
    grid=(N // ABLK,),
    in_specs=[pl.BlockSpec((ABLK, D), lambda i: (i, 0))] * 5,
    out_specs=pl.BlockSpec((ABLK, D), lambda i: (i, 0)),
    out_shape=jax.ShapeDtypeStruct((N, D), jnp.float32),
)


def _l2_body(x, out):
    v = x[...]
    out[...] = jnp.sum(jnp.sum(v * v, axis=2), axis=0, keepdims=True)


_tc_l2 = pl.pallas_call(
    _l2_body,
    out_shape=jax.ShapeDtypeStruct((1, B), jnp.float32),
)


def kernel(embedding, wq, bq, wk, bk, wv, bv, w_out_w, w_out_b, feat_index,
           feat_values, adj_index, adj_values, sampled_users, sampled_items,
           users, pos_items, neg_items):
    f32 = jnp.float32
    i32 = jnp.int32
    emb = embedding.astype(f32)

    def _pack(x, dt):
        xp = jnp.concatenate([x.astype(dt), jnp.zeros((E_PAD - E,), dt)])
        a = xp[:NS * K0 * C].reshape(NS, K0, C)
        b = xp[NS * K0 * C:].reshape(NS, K1, C)
        a = jnp.pad(a, ((0, 0), (0, NCH - K0), (0, 0)))
        b = jnp.pad(b, ((0, 0), (0, NCH - K1), (0, 0)))
        return jnp.stack([a, b], axis=1).reshape(NW, NCH, C)

    def _pad_i(x):
        return _pack(x, i32)

    def _pad_f(x):
        return _pack(x, f32)

    fr = _pad_i(feat_index[0])
    fc = _pad_i(feat_index[1])
    fv = _pad_f(feat_values)
    ar = _pad_i(adj_index[0])
    ac = _pad_i(adj_index[1])
    av = _pad_f(adj_values)

    xq_p = _sc_spmm(emb, fr, fc, fv)

    nb_idx = jnp.concatenate([
        sampled_users.reshape(-1).astype(i32),
        sampled_items.reshape(-1).astype(i32) + N_USERS,
        jnp.zeros((NW * 16 - 2 * H * S,), i32),
    ])
    nb_rows = _sc_gather_nb(emb, nb_idx)
    nb = jnp.stack([nb_rows[:H * S].reshape(H, S, D),
                    nb_rows[H * S:2 * H * S].reshape(H, S, D)], axis=1)

    wqk, cb, nvo, bt = _tc_pre(nb, wq.astype(f32), bq.astype(f32),
                               wk.astype(f32), bk.astype(f32), wv.astype(f32),
                               bv.astype(f32), w_out_w.astype(f32),
                               w_out_b.astype(f32).reshape(1, D))

    reps = _tc_gat(xq_p[:N], xq_p[NP:NP + N], wqk, cb, nvo, bt)

    p1 = _sc_spmm(reps, ar, ac, av)
    cur1 = _tc_add(p1[:N], p1[NP:NP + N])
    p2 = _sc_spmm(cur1, ar, ac, av)
    cur2 = _tc_add(p2[:N], p2[NP:NP + N])
    p3 = _sc_spmm(cur2, ar, ac, av)
    finalr = _tc_mean(reps, cur1, cur2, p3[:N], p3[NP:NP + N])

    gidx = jnp.concatenate([
        users.astype(i32),
        pos_items.astype(i32) + N_USERS,
        neg_items.astype(i32) + N_USERS,
    ])
    rep_g = _sc_gather_bpr(finalr, gidx)
    emb_g = _sc_gather_bpr(emb, gidx)
    l2 = _tc_l2(emb_g.reshape(3, B, D))[0]

    return rep_g[:B], rep_g[B:2 * B], rep_g[2 * B:], l2


# speedup vs baseline: 1.3132x; 1.0538x over previous
import functools

import jax
import jax.numpy as jnp
from jax import lax
from jax.experimental import pallas as pl
from jax.experimental.pallas import tpu as pltpu
from jax.experimental.pallas import tpu_sc as plsc

N_USERS = 5000
N_ITEMS = 5000
N = 10000
D = 128
E = 320000
H = 4
S = 50
B = 4096

NC = 2
NS = 16
NW = NC * NS
L = 16

C = 128
K0 = 72
K1 = 85
NCH = 112
NPASS = 2
NCHP = NCH // NPASS
E_PAD = NS * (K0 + K1) * C
NP = 10112
RPT = NP // NS

_mesh = plsc.VectorSubcoreMesh(core_axis_name="c", subcore_axis_name="s",
                               num_cores=NC, num_subcores=NS)


def _spmm_body(src, rows_i, cols_i, vals, out, acc, idx_r, idx_c, vv, rb0,
               sg0):
    cid = lax.axis_index("c")
    sid = lax.axis_index("s")
    wid = sid * NC + cid

    def zrow(i, carry):
        for k in range(D // L):
            rb0[i, pl.ds(k * L, L)] = jnp.zeros((L,), jnp.float32)
        return carry
    lax.fori_loop(0, C, zrow, 0)
    r0 = sid * RPT
    for q in range(RPT // C):
        pltpu.sync_copy(rb0, acc.at[pl.ds(r0 + q * C, C)])
    rem = RPT - (RPT // C) * C
    if rem:
        pltpu.sync_copy(rb0.at[pl.ds(0, rem)],
                        acc.at[pl.ds(r0 + (RPT // C) * C, rem)])
    plsc.subcore_barrier()

    nch_me = jnp.where(cid == 0, K0, K1)
    for p in range(NPASS):
        pltpu.sync_copy(rows_i.at[wid, pl.ds(p * NCHP, NCHP)], idx_r)
        pltpu.sync_copy(cols_i.at[wid, pl.ds(p * NCHP, NCHP)], idx_c)
        pltpu.sync_copy(vals.at[wid, pl.ds(p * NCHP, NCHP)], vv)
        cnt = jnp.minimum(jnp.maximum(nch_me - p * NCHP, 0), NCHP)

        def chunk(j, carry):
            pltpu.async_copy(src.at[idx_c.at[j]], rb0, sg0).wait()

            def scale16(g, c2):
                vchunk = vv[j, pl.ds(g * L, L)]
                for e16 in range(L):
                    e = g * L + e16
                    val = jnp.broadcast_to(vchunk[e16], (L,))
                    for k in range(D // L):
                        sl = (e, pl.ds(k * L, L))
                        rb0[sl] = rb0[sl] * val
                return c2
            lax.fori_loop(0, C // L, scale16, 0)
            pltpu.sync_copy(rb0, acc.at[idx_r.at[j]], add=True)
            return carry
        lax.fori_loop(0, cnt, chunk, 0)

    plsc.subcore_barrier()
    pltpu.sync_copy(acc.at[pl.ds(r0, RPT)], out.at[pl.ds(cid * NP + r0, RPT)])


_sc_spmm = pl.kernel(
    _spmm_body,
    out_type=jax.ShapeDtypeStruct((NC * NP, D), jnp.float32),
    mesh=_mesh,
    scratch_types=[
        pltpu.VMEM_SHARED((NP, D), jnp.float32),
        pltpu.VMEM((NCHP, C), jnp.int32),
        pltpu.VMEM((NCHP, C), jnp.int32),
        pltpu.VMEM((NCHP, C), jnp.float32),
        pltpu.VMEM((C, D), jnp.float32),
        pltpu.SemaphoreType.DMA,
    ],
)


def _make_sc_gather(K, CG):
    def body(src, gidx, out, idxv, buf, sem):
        cid = lax.axis_index("c")
        sid = lax.axis_index("s")
        wid = sid * NC + cid
        for t in range(K):
            pltpu.sync_copy(gidx.at[pl.ds((wid * K + t) * CG, CG)], idxv)
            pltpu.async_copy(src.at[idxv], buf, sem).wait()
            pltpu.sync_copy(buf, out.at[pl.ds((wid * K + t) * CG, CG)])

    return pl.kernel(
        body,
        out_type=jax.ShapeDtypeStruct((NW * K * CG, D), jnp.float32),
        mesh=_mesh,
        scratch_types=[
            pltpu.VMEM((CG,), jnp.int32),
            pltpu.VMEM((CG, D), jnp.float32),
            pltpu.SemaphoreType.DMA,
        ],
    )


_sc_gather_nb = _make_sc_gather(1, 16)
_sc_gather_bpr = _make_sc_gather(3, 128)


def _pre_body(nb, wq, bq, wk, bk, wv, bv, wo, wob, wqk_o, cb_o, nvo_o, bt_o):
    f32 = jnp.float32
    bt = wob[...]
    for h in range(H):
        woh = wo[:, h * D:(h + 1) * D]
        bt = bt + lax.dot_general(bv[h][None, :], woh,
                                  (((1,), (1,)), ((), ())),
                                  preferred_element_type=f32)
        for s2 in range(2):
            nbs = nb[h, s2]
            khat = lax.dot_general(nbs, wk[h], (((1,), (1,)), ((), ())),
                                   preferred_element_type=f32) + bk[h][None, :]
            wqk_o[h, s2] = lax.dot_general(wq[h], khat,
                                           (((0,), (1,)), ((), ())),
                                           preferred_element_type=f32)
            cb_o[h, s2] = lax.dot_general(bq[h][None, :], khat,
                                          (((1,), (1,)), ((), ())),
                                          preferred_element_type=f32)
            nv = lax.dot_general(nbs, wv[h], (((1,), (1,)), ((), ())),
                                 preferred_element_type=f32) + bv[h][None, :]
            nvo_o[h, s2] = lax.dot_general(nv, woh, (((1,), (1,)), ((), ())),
                                           preferred_element_type=f32)
    bt_o[...] = bt


_tc_pre = pl.pallas_call(
    _pre_body,
    out_shape=[
        jax.ShapeDtypeStruct((H, 2, D, S), jnp.float32),
        jax.ShapeDtypeStruct((H, 2, 1, S), jnp.float32),
        jax.ShapeDtypeStruct((H, 2, S, D), jnp.float32),
        jax.ShapeDtypeStruct((1, D), jnp.float32),
    ],
)


BLK = 1000
NBLK = N_USERS // BLK


def _gat_body(x0, x1, wqk, cb, nvo, bt, out):
    f32 = jnp.float32
    x = x0[...] + x1[...]
    acc = jnp.broadcast_to(bt[...], (BLK, D))
    for h in range(H):
        logits = jnp.dot(x, wqk[h, 0], preferred_element_type=f32)
        logits = logits + cb[h, 0, 0][None, :]
        m = jnp.max(logits, axis=1, keepdims=True)
        p = jnp.exp(logits - m)
        attn = p / jnp.sum(p, axis=1, keepdims=True)
        acc = acc + jnp.dot(attn, nvo[h, 0], preferred_element_type=f32)
    out[...] = acc


_tc_gat = pl.pallas_call(
    _gat_body,
    grid=(2, NBLK),
    in_specs=[
        pl.BlockSpec((BLK, D), lambda u, b: (u * NBLK + b, 0)),
        pl.BlockSpec((BLK, D), lambda u, b: (u * NBLK + b, 0)),
        pl.BlockSpec((H, 1, D, S), lambda u, b: (0, u, 0, 0)),
        pl.BlockSpec((H, 1, 1, S), lambda u, b: (0, u, 0, 0)),
        pl.BlockSpec((H, 1, S, D), lambda u, b: (0, u, 0, 0)),
        pl.BlockSpec((1, D), lambda u, b: (0, 0)),
    ],
    out_specs=pl.BlockSpec((BLK, D), lambda u, b: (u * NBLK + b, 0)),
    out_shape=jax.ShapeDtypeStruct((N, D), jnp.float32),
)


ABLK = 2000


def _add_body(a, b, out):
    out[...] = a[...] + b[...]


_tc_add = pl.pallas_call(
    _add_body,
    grid=(N // ABLK,),
    in_specs=[pl.BlockSpec((ABLK, D), lambda i: (i, 0))] * 2,
    out_specs=pl.BlockSpec((ABLK, D), lambda i: (i, 0)),
    out_shape=jax.ShapeDtypeStruct((N, D), jnp.float32),
)


def _mean_body(a, b, c, d, e, out):
    out[...] = (a[...] + b[...] + c[...] + d[...] + e[...]) * 0.25


_tc_mean = pl.pallas_call(
    _mean_body,
    grid=(N // ABLK,),
    in_specs=[pl.BlockSpec((ABLK, D), lambda i: (i, 0))] * 5,
    out_specs=pl.BlockSpec((ABLK, D), lambda i: (i, 0)),
    out_shape=jax.ShapeDtypeStruct((N, D), jnp.float32),
)


def _l2_body(x, out):
    v = x[...]
    out[...] = jnp.sum(jnp.sum(v * v, axis=2), axis=0, keepdims=True)


_tc_l2 = pl.pallas_call(
    _l2_body,
    out_shape=jax.ShapeDtypeStruct((1, B), jnp.float32),
)


def kernel(embedding, wq, bq, wk, bk, wv, bv, w_out_w, w_out_b, feat_index,
           feat_values, adj_index, adj_values, sampled_users, sampled_items,
           users, pos_items, neg_items):
    f32 = jnp.float32
    i32 = jnp.int32
    emb = embedding.astype(f32)

    def _pack(x, dt):
        xp = jnp.concatenate([x.astype(dt), jnp.zeros((E_PAD - E,), dt)])
        a = xp[:NS * K0 * C].reshape(NS, K0, C)
        b = xp[NS * K0 * C:].reshape(NS, K1, C)
        a = jnp.pad(a, ((0, 0), (0, NCH - K0), (0, 0)))
        b = jnp.pad(b, ((0, 0), (0, NCH - K1), (0, 0)))
        return jnp.stack([a, b], axis=1).reshape(NW, NCH, C)

    def _pad_i(x):
        return _pack(x, i32)

    def _pad_f(x):
        return _pack(x, f32)

    fr = _pad_i(feat_index[0])
    fc = _pad_i(feat_index[1])
    fv = _pad_f(feat_values)
    ar = _pad_i(adj_index[0])
    ac = _pad_i(adj_index[1])
    av = _pad_f(adj_values)

    xq_p = _sc_spmm(emb, fr, fc, fv)

    nb_idx = jnp.concatenate([
        sampled_users.reshape(-1).astype(i32),
        sampled_items.reshape(-1).astype(i32) + N_USERS,
        jnp.zeros((NW * 16 - 2 * H * S,), i32),
    ])
    nb_rows = _sc_gather_nb(emb, nb_idx)
    nb = jnp.stack([nb_rows[:H * S].reshape(H, S, D),
                    nb_rows[H * S:2 * H * S].reshape(H, S, D)], axis=1)

    wqk, cb, nvo, bt = _tc_pre(nb, wq.astype(f32), bq.astype(f32),
                               wk.astype(f32), bk.astype(f32), wv.astype(f32),
                               bv.astype(f32), w_out_w.astype(f32),
                               w_out_b.astype(f32).reshape(1, D))

    reps = _tc_gat(xq_p[:N], xq_p[NP:NP + N], wqk, cb, nvo, bt)

    p1 = _sc_spmm(reps, ar, ac, av)
    cur1 = _tc_add(p1[:N], p1[NP:NP + N])
    p2 = _sc_spmm(cur1, ar, ac, av)
    cur2 = _tc_add(p2[:N], p2[NP:NP + N])
    p3 = _sc_spmm(cur2, ar, ac, av)
    finalr = _tc_mean(reps, cur1, cur2, p3[:N], p3[NP:NP + N])

    gidx = jnp.concatenate([
        users.astype(i32),
        pos_items.astype(i32) + N_USERS,
        neg_items.astype(i32) + N_USERS,
    ])
    rep_g = _sc_gather_bpr(finalr, gidx)
    emb_g = _sc_gather_bpr(emb, gidx)
    l2 = _tc_l2(emb_g.reshape(3, B, D))[0]

    return rep_g[:B], rep_g[B:2 * B], rep_g[2 * B:], l2


# speedup vs baseline: 1.3654x; 1.0398x over previous
import functools

import jax
import jax.numpy as jnp
from jax import lax
from jax.experimental import pallas as pl
from jax.experimental.pallas import tpu as pltpu
from jax.experimental.pallas import tpu_sc as plsc

N_USERS = 5000
N_ITEMS = 5000
N = 10000
D = 128
E = 320000
H = 4
S = 50
B = 4096

NC = 2
NS = 16
NW = NC * NS
L = 16

C = 128
K0 = 79
K1 = 78
NCH = 112
NPASS = 2
NCHP = NCH // NPASS
E_PAD = NS * (K0 + K1) * C
NP = 10112
RPT = NP // NS

_mesh = plsc.VectorSubcoreMesh(core_axis_name="c", subcore_axis_name="s",
                               num_cores=NC, num_subcores=NS)


def _spmm_body(src, rows_i, cols_i, vals, out, acc, idx_r, idx_c, vv, rb0,
               sg0):
    cid = lax.axis_index("c")
    sid = lax.axis_index("s")
    wid = sid * NC + cid

    def zrow(i, carry):
        for k in range(D // L):
            rb0[i, pl.ds(k * L, L)] = jnp.zeros((L,), jnp.float32)
        return carry
    lax.fori_loop(0, C, zrow, 0)
    r0 = sid * RPT
    for q in range(RPT // C):
        pltpu.sync_copy(rb0, acc.at[pl.ds(r0 + q * C, C)])
    rem = RPT - (RPT // C) * C
    if rem:
        pltpu.sync_copy(rb0.at[pl.ds(0, rem)],
                        acc.at[pl.ds(r0 + (RPT // C) * C, rem)])
    plsc.subcore_barrier()

    nch_me = jnp.where(cid == 0, K0, K1)
    for p in range(NPASS):
        pltpu.sync_copy(rows_i.at[wid, pl.ds(p * NCHP, NCHP)], idx_r)
        pltpu.sync_copy(cols_i.at[wid, pl.ds(p * NCHP, NCHP)], idx_c)
        pltpu.sync_copy(vals.at[wid, pl.ds(p * NCHP, NCHP)], vv)
        cnt = jnp.minimum(jnp.maximum(nch_me - p * NCHP, 0), NCHP)

        def chunk(j, carry):
            pltpu.async_copy(src.at[idx_c.at[j]], rb0, sg0).wait()

            def scale16(g, c2):
                vchunk = vv[j, pl.ds(g * L, L)]
                for e16 in range(L):
                    e = g * L + e16
                    val = jnp.broadcast_to(vchunk[e16], (L,))
                    for k in range(D // L):
                        sl = (e, pl.ds(k * L, L))
                        rb0[sl] = rb0[sl] * val
                return c2
            lax.fori_loop(0, C // L, scale16, 0)
            pltpu.sync_copy(rb0, acc.at[idx_r.at[j]], add=True)
            return carry
        lax.fori_loop(0, cnt, chunk, 0)

    plsc.subcore_barrier()
    pltpu.sync_copy(acc.at[pl.ds(r0, RPT)], out.at[pl.ds(cid * NP + r0, RPT)])


_sc_spmm = pl.kernel(
    _spmm_body,
    out_type=jax.ShapeDtypeStruct((NC * NP, D), jnp.float32),
    mesh=_mesh,
    scratch_types=[
        pltpu.VMEM_SHARED((NP, D), jnp.float32),
        pltpu.VMEM((NCHP, C), jnp.int32),
        pltpu.VMEM((NCHP, C), jnp.int32),
        pltpu.VMEM((NCHP, C), jnp.float32),
        pltpu.VMEM((C, D), jnp.float32),
        pltpu.SemaphoreType.DMA,
    ],
)


def _make_sc_gather(K, CG):
    def body(src, gidx, out, idxv, buf, sem):
        cid = lax.axis_index("c")
        sid = lax.axis_index("s")
        wid = sid * NC + cid
        for t in range(K):
            pltpu.sync_copy(gidx.at[pl.ds((wid * K + t) * CG, CG)], idxv)
            pltpu.async_copy(src.at[idxv], buf, sem).wait()
            pltpu.sync_copy(buf, out.at[pl.ds((wid * K + t) * CG, CG)])

    return pl.kernel(
        body,
        out_type=jax.ShapeDtypeStruct((NW * K * CG, D), jnp.float32),
        mesh=_mesh,
        scratch_types=[
            pltpu.VMEM((CG,), jnp.int32),
            pltpu.VMEM((CG, D), jnp.float32),
            pltpu.SemaphoreType.DMA,
        ],
    )


_sc_gather_nb = _make_sc_gather(1, 16)
_sc_gather_bpr = _make_sc_gather(3, 128)


def _pre_body(nb, wq, bq, wk, bk, wv, bv, wo, wob, wqk_o, cb_o, nvo_o, bt_o):
    f32 = jnp.float32
    bt = wob[...]
    for h in range(H):
        woh = wo[:, h * D:(h + 1) * D]
        bt = bt + lax.dot_general(bv[h][None, :], woh,
                                  (((1,), (1,)), ((), ())),
                                  preferred_element_type=f32)
        for s2 in range(2):
            nbs = nb[h, s2]
            khat = lax.dot_general(nbs, wk[h], (((1,), (1,)), ((), ())),
                                   preferred_element_type=f32) + bk[h][None, :]
            wqk_o[h, s2] = lax.dot_general(wq[h], khat,
                                           (((0,), (1,)), ((), ())),
                                           preferred_element_type=f32)
            cb_o[h, s2] = lax.dot_general(bq[h][None, :], khat,
                                          (((1,), (1,)), ((), ())),
                                          preferred_element_type=f32)
            nv = lax.dot_general(nbs, wv[h], (((1,), (1,)), ((), ())),
                                 preferred_element_type=f32) + bv[h][None, :]
            nvo_o[h, s2] = lax.dot_general(nv, woh, (((1,), (1,)), ((), ())),
                                           preferred_element_type=f32)
    bt_o[...] = bt


_tc_pre = pl.pallas_call(
    _pre_body,
    out_shape=[
        jax.ShapeDtypeStruct((H, 2, D, S), jnp.float32),
        jax.ShapeDtypeStruct((H, 2, 1, S), jnp.float32),
        jax.ShapeDtypeStruct((H, 2, S, D), jnp.float32),
        jax.ShapeDtypeStruct((1, D), jnp.float32),
    ],
)


BLK = 1000
NBLK = N_USERS // BLK


def _gat_body(x0, x1, wqk, cb, nvo, bt, out):
    f32 = jnp.float32
    x = x0[...] + x1[...]
    acc = jnp.broadcast_to(bt[...], (BLK, D))
    for h in range(H):
        logits = jnp.dot(x, wqk[h, 0], preferred_element_type=f32)
        logits = logits + cb[h, 0, 0][None, :]
        m = jnp.max(logits, axis=1, keepdims=True)
        p = jnp.exp(logits - m)
        attn = p / jnp.sum(p, axis=1, keepdims=True)
        acc = acc + jnp.dot(attn, nvo[h, 0], preferred_element_type=f32)
    out[...] = acc


_tc_gat = pl.pallas_call(
    _gat_body,
    grid=(2, NBLK),
    in_specs=[
        pl.BlockSpec((BLK, D), lambda u, b: (u * NBLK + b, 0)),
        pl.BlockSpec((BLK, D), lambda u, b: (u * NBLK + b, 0)),
        pl.BlockSpec((H, 1, D, S), lambda u, b: (0, u, 0, 0)),
        pl.BlockSpec((H, 1, 1, S), lambda u, b: (0, u, 0, 0)),
        pl.BlockSpec((H, 1, S, D), lambda u, b: (0, u, 0, 0)),
        pl.BlockSpec((1, D), lambda u, b: (0, 0)),
    ],
    out_specs=pl.BlockSpec((BLK, D), lambda u, b: (u * NBLK + b, 0)),
    out_shape=jax.ShapeDtypeStruct((N, D), jnp.float32),
)


ABLK = 2000


def _add_body(a, b, out):
    out[...] = a[...] + b[...]


_tc_add = pl.pallas_call(
    _add_body,
    grid=(N // ABLK,),
    in_specs=[pl.BlockSpec((ABLK, D), lambda i: (i, 0))] * 2,
    out_specs=pl.BlockSpec((ABLK, D), lambda i: (i, 0)),
    out_shape=jax.ShapeDtypeStruct((N, D), jnp.float32),
)


def _mean_body(a, b, c, d, e, out):
    out[...] = (a[...] + b[...] + c[...] + d[...] + e[...]) * 0.25


_tc_mean = pl.pallas_call(
    _mean_body,
    grid=(N // ABLK,),
    in_specs=[pl.BlockSpec((ABLK, D), lambda i: (i, 0))] * 5,
    out_specs=pl.BlockSpec((ABLK, D), lambda i: (i, 0)),
    out_shape=jax.ShapeDtypeStruct((N, D), jnp.float32),
)


def _l2_body(x, out):
    v = x[...]
    out[...] = jnp.sum(jnp.sum(v * v, axis=2), axis=0, keepdims=True)


_tc_l2 = pl.pallas_call(
    _l2_body,
    out_shape=jax.ShapeDtypeStruct((1, B), jnp.float32),
)


def kernel(embedding, wq, bq, wk, bk, wv, bv, w_out_w, w_out_b, feat_index,
           feat_values, adj_index, adj_values, sampled_users, sampled_items,
           users, pos_items, neg_items):
    f32 = jnp.float32
    i32 = jnp.int32
    emb = embedding.astype(f32)

    def _pack(x, dt):
        xp = jnp.concatenate([x.astype(dt), jnp.zeros((E_PAD - E,), dt)])
        a = xp[:NS * K0 * C].reshape(NS, K0, C)
        b = xp[NS * K0 * C:].reshape(NS, K1, C)
        a = jnp.pad(a, ((0, 0), (0, NCH - K0), (0, 0)))
        b = jnp.pad(b, ((0, 0), (0, NCH - K1), (0, 0)))
        return jnp.stack([a, b], axis=1).reshape(NW, NCH, C)

    def _pad_i(x):
        return _pack(x, i32)

    def _pad_f(x):
        return _pack(x, f32)

    fr = _pad_i(feat_index[0])
    fc = _pad_i(feat_index[1])
    fv = _pad_f(feat_values)
    ar = _pad_i(adj_index[0])
    ac = _pad_i(adj_index[1])
    av = _pad_f(adj_values)

    xq_p = _sc_spmm(emb, fr, fc, fv)

    nb_idx = jnp.concatenate([
        sampled_users.reshape(-1).astype(i32),
        sampled_items.reshape(-1).astype(i32) + N_USERS,
        jnp.zeros((NW * 16 - 2 * H * S,), i32),
    ])
    nb_rows = _sc_gather_nb(emb, nb_idx)
    nb = jnp.stack([nb_rows[:H * S].reshape(H, S, D),
                    nb_rows[H * S:2 * H * S].reshape(H, S, D)], axis=1)

    wqk, cb, nvo, bt = _tc_pre(nb, wq.astype(f32), bq.astype(f32),
                               wk.astype(f32), bk.astype(f32), wv.astype(f32),
                               bv.astype(f32), w_out_w.astype(f32),
                               w_out_b.astype(f32).reshape(1, D))

    reps = _tc_gat(xq_p[:N], xq_p[NP:NP + N], wqk, cb, nvo, bt)

    p1 = _sc_spmm(reps, ar, ac, av)
    cur1 = _tc_add(p1[:N], p1[NP:NP + N])
    p2 = _sc_spmm(cur1, ar, ac, av)
    cur2 = _tc_add(p2[:N], p2[NP:NP + N])
    p3 = _sc_spmm(cur2, ar, ac, av)
    finalr = _tc_mean(reps, cur1, cur2, p3[:N], p3[NP:NP + N])

    gidx = jnp.concatenate([
        users.astype(i32),
        pos_items.astype(i32) + N_USERS,
        neg_items.astype(i32) + N_USERS,
    ])
    rep_g = _sc_gather_bpr(finalr, gidx)
    emb_g = _sc_gather_bpr(emb, gidx)
    l2 = _tc_l2(emb_g.reshape(3, B, D))[0]

    return rep_g[:B], rep_g[B:2 * B], rep_g[2 * B:], l2
